# Initial kernel scaffold; baseline (speedup 1.0000x reference)
import jax
import jax.numpy as jnp
from jax.experimental import pallas as pl


def kernel(x, edge_index, edge_weights):
    raise NotImplementedError("write your pallas kernel here")


# speedup vs baseline: 30.3027x; 30.3027x over previous
import functools

import jax
import jax.numpy as jnp
from jax import lax
from jax.experimental import pallas as pl
from jax.experimental.pallas import tpu as pltpu
from jax.experimental.pallas import tpu_sc as plsc

N = 10000
E = 320000
D = 128

NPAD = 10240
NROWS = NPAD // 128
NC = 2
NS = 16
NW = NC * NS
EPT = E // NW
CHUNK = 128
ROWS = (EPT + CHUNK - 1) // CHUNK
EPT_PAD = ROWS * CHUNK
SLICE = NPAD // NS


def _rowsum_body(x_ref, o_ref):
    o_ref[...] = jnp.sum(x_ref[...], axis=1).reshape(8, 128)


_rowsum = pl.pallas_call(
    _rowsum_body,
    grid=(NPAD // 1024,),
    in_specs=[pl.BlockSpec((1024, D), lambda i: (i, 0))],
    out_specs=pl.BlockSpec((8, 128), lambda i: (i, 0)),
    out_shape=jax.ShapeDtypeStruct((NROWS, 128), jnp.float32),
)


def _edge_body(s_hbm, src_hbm, dst_hbm, w_hbm, valid_hbm,
               t_out, c_out,
               s_v, src_v, w_v, valid_v, msg_v, dst_v, zb_v,
               t_sh, c_sh, sem):
    cid = lax.axis_index("c")
    sid = lax.axis_index("s")
    wid = sid * NC + cid

    pltpu.sync_copy(s_hbm, s_v)
    pltpu.sync_copy(src_hbm.at[wid], src_v)
    pltpu.sync_copy(w_hbm.at[wid], w_v)
    pltpu.sync_copy(valid_hbm, valid_v)
    pltpu.sync_copy(dst_hbm.at[wid], dst_v)

    def zbody(i, c):
        zb_v[pl.ds(i * 16, 16)] = jnp.zeros((16,), jnp.float32)
        return c
    lax.fori_loop(0, SLICE // 16, zbody, 0)
    pltpu.sync_copy(zb_v, t_sh.at[pl.ds(sid * SLICE, SLICE)])
    pltpu.sync_copy(zb_v, c_sh.at[pl.ds(sid * SLICE, SLICE)])
    plsc.subcore_barrier()

    def mbody(i, c):
        sl = pl.ds(i * 16, 16)
        vals = plsc.load_gather(s_v, [src_v[sl]])
        msg_v[sl] = vals * w_v[sl]
        return c
    lax.fori_loop(0, EPT_PAD // 16, mbody, 0)

    def sbody(j, c):
        row = pl.ds(j * CHUNK, CHUNK)
        cp1 = pltpu.async_copy(msg_v.at[row], t_sh.at[dst_v.at[j]], sem,
                               add=True)
        cp2 = pltpu.async_copy(valid_v.at[row], c_sh.at[dst_v.at[j]], sem,
                               add=True)
        cp1.wait()
        cp2.wait()
        return c
    lax.fori_loop(0, ROWS, sbody, 0)
    plsc.subcore_barrier()

    off = sid * SLICE
    pltpu.sync_copy(t_sh.at[pl.ds(off, SLICE)], t_out.at[cid, pl.ds(off, SLICE)])
    pltpu.sync_copy(c_sh.at[pl.ds(off, SLICE)], c_out.at[cid, pl.ds(off, SLICE)])


_edge_call = pl.kernel(
    _edge_body,
    out_type=[jax.ShapeDtypeStruct((NC, NPAD), jnp.float32),
              jax.ShapeDtypeStruct((NC, NPAD), jnp.float32)],
    mesh=plsc.VectorSubcoreMesh(core_axis_name="c", subcore_axis_name="s",
                                num_cores=NC, num_subcores=NS),
    scratch_types=[
        pltpu.VMEM((NPAD,), jnp.float32),
        pltpu.VMEM((EPT_PAD,), jnp.int32),
        pltpu.VMEM((EPT_PAD,), jnp.float32),
        pltpu.VMEM((EPT_PAD,), jnp.float32),
        pltpu.VMEM((EPT_PAD,), jnp.float32),
        pltpu.VMEM((ROWS, CHUNK), jnp.int32),
        pltpu.VMEM((SLICE,), jnp.float32),
        pltpu.VMEM_SHARED((NPAD,), jnp.float32),
        pltpu.VMEM_SHARED((NPAD,), jnp.float32),
        pltpu.SemaphoreType.DMA,
    ],
    compiler_params=pltpu.CompilerParams(needs_layout_passes=False),
)


def _final_body(s_ref, t_ref, c_ref, o_ref):
    t = t_ref[0] + t_ref[1]
    c = jnp.maximum(c_ref[0] + c_ref[1], 1.0)
    o_ref[...] = jnp.abs(s_ref[...] - t / c)


_final = pl.pallas_call(
    _final_body,
    out_shape=jax.ShapeDtypeStruct((NROWS, 128), jnp.float32),
)


def kernel(x, edge_index, edge_weights):
    x_pad = jnp.pad(x, ((0, NPAD - N), (0, 0)))
    s2d = _rowsum(x_pad)
    s_flat = s2d.reshape(NPAD)

    pad = ((0, 0), (0, EPT_PAD - EPT))
    src_p = jnp.pad(edge_index[0].reshape(NW, EPT), pad)
    dst_p = jnp.pad(edge_index[1].reshape(NW, EPT), pad).reshape(NW, ROWS, CHUNK)
    w_p = jnp.pad(edge_weights.reshape(NW, EPT), pad)
    valid = (jnp.arange(EPT_PAD, dtype=jnp.int32) < EPT).astype(jnp.float32)

    t_part, c_part = _edge_call(s_flat, src_p, dst_p, w_p, valid)

    info2d = _final(s2d, t_part.reshape(NC, NROWS, 128),
                    c_part.reshape(NC, NROWS, 128))
    return info2d.reshape(NPAD)[:N]


# speedup vs baseline: 36.9155x; 1.2182x over previous
import jax
import jax.numpy as jnp
from jax import lax
from jax.experimental import pallas as pl
from jax.experimental.pallas import tpu as pltpu
from jax.experimental.pallas import tpu_sc as plsc

N = 10000
E = 320000
D = 128

NPAD = 10240
NROWS = NPAD // 128
NC = 2
NS = 16
NW = NC * NS
EPT = E // NW
CHUNK = 80
ROWS = EPT // CHUNK
GRP = 5
NGRP = ROWS // GRP
SLICE = NPAD // NS


def _rowsum_body(x_ref, o_ref):
    ones = jnp.ones((D, 1), jnp.float32)
    o_ref[...] = jnp.dot(x_ref[...], ones, preferred_element_type=jnp.float32,
                         precision=jax.lax.Precision.HIGHEST)


_rowsum = pl.pallas_call(
    _rowsum_body,
    grid=(NPAD // 1024,),
    in_specs=[pl.BlockSpec((1024, D), lambda i: (i, 0))],
    out_specs=pl.BlockSpec((1024, 1), lambda i: (i, 0)),
    out_shape=jax.ShapeDtypeStruct((NPAD, 1), jnp.float32),
)


def _edge_body(s_hbm, src_hbm, dst_hbm, w_hbm, t_out, c_out,
               s_v, src_v, w_v, msg_v, dst_v, ones_v, zb_v,
               t_sh, c_sh, sem, scat_sem):
    cid = lax.axis_index("c")
    sid = lax.axis_index("s")
    wid = sid * NC + cid
    base = wid * EPT

    cps = [
        pltpu.async_copy(s_hbm, s_v, sem),
        pltpu.async_copy(src_hbm.at[pl.ds(base, EPT)], src_v, sem),
        pltpu.async_copy(w_hbm.at[pl.ds(base, EPT)], w_v, sem),
        pltpu.async_copy(dst_hbm.at[wid], dst_v, sem),
    ]

    for i in range(CHUNK // 16):
        ones_v[pl.ds(i * 16, 16)] = jnp.ones((16,), jnp.float32)

    def zbody(i, c):
        zb_v[pl.ds(i * 16, 16)] = jnp.zeros((16,), jnp.float32)
        return c
    lax.fori_loop(0, SLICE // 16, zbody, 0)
    pltpu.sync_copy(zb_v, t_sh.at[pl.ds(sid * SLICE, SLICE)])
    pltpu.sync_copy(zb_v, c_sh.at[pl.ds(sid * SLICE, SLICE)])
    for cp in cps:
        cp.wait()
    plsc.subcore_barrier()

    def scat_pair(j):
        row = pl.ds(j * CHUNK, CHUNK)
        c1 = pltpu.make_async_copy(msg_v.at[row], t_sh.at[dst_v.at[j]],
                                   scat_sem)
        c2 = pltpu.make_async_copy(ones_v, c_sh.at[dst_v.at[j]], scat_sem)
        return c1, c2

    def gbody(g, c):
        for r in range(GRP):
            j = g * GRP + r
            for k in range(CHUNK // 16):
                sl = pl.ds(j * CHUNK + k * 16, 16)
                vals = plsc.load_gather(s_v, [src_v[sl]])
                msg_v[sl] = vals * w_v[sl]
        for r in range(GRP):
            c1, c2 = scat_pair(g * GRP + r)
            c1.start(add=True)
            c2.start(add=True)

        @pl.when(g > 0)
        def _():
            for r in range(GRP):
                c1, c2 = scat_pair((g - 1) * GRP + r)
                c1.wait()
                c2.wait()
        return c
    lax.fori_loop(0, NGRP, gbody, 0)
    for r in range(GRP):
        c1, c2 = scat_pair((NGRP - 1) * GRP + r)
        c1.wait()
        c2.wait()
    plsc.subcore_barrier()

    off = sid * SLICE
    cpo1 = pltpu.async_copy(t_sh.at[pl.ds(off, SLICE)],
                            t_out.at[cid, pl.ds(off, SLICE)], sem)
    cpo2 = pltpu.async_copy(c_sh.at[pl.ds(off, SLICE)],
                            c_out.at[cid, pl.ds(off, SLICE)], sem)
    cpo1.wait()
    cpo2.wait()


_edge_call = pl.kernel(
    _edge_body,
    out_type=[jax.ShapeDtypeStruct((NC, NPAD), jnp.float32),
              jax.ShapeDtypeStruct((NC, NPAD), jnp.float32)],
    mesh=plsc.VectorSubcoreMesh(core_axis_name="c", subcore_axis_name="s",
                                num_cores=NC, num_subcores=NS),
    scratch_types=[
        pltpu.VMEM((NPAD,), jnp.float32),
        pltpu.VMEM((EPT,), jnp.int32),
        pltpu.VMEM((EPT,), jnp.float32),
        pltpu.VMEM((EPT,), jnp.float32),
        pltpu.VMEM((ROWS, CHUNK), jnp.int32),
        pltpu.VMEM((CHUNK,), jnp.float32),
        pltpu.VMEM((SLICE,), jnp.float32),
        pltpu.VMEM_SHARED((NPAD,), jnp.float32),
        pltpu.VMEM_SHARED((NPAD,), jnp.float32),
        pltpu.SemaphoreType.DMA,
        pltpu.SemaphoreType.DMA,
    ],
    compiler_params=pltpu.CompilerParams(needs_layout_passes=False),
)


def _final_body(s_ref, t_ref, c_ref, o_ref):
    t = t_ref[0] + t_ref[1]
    c = jnp.maximum(c_ref[0] + c_ref[1], 1.0)
    o_ref[...] = jnp.abs(s_ref[...] - t / c)


_final = pl.pallas_call(
    _final_body,
    out_shape=jax.ShapeDtypeStruct((NROWS, 128), jnp.float32),
)


def kernel(x, edge_index, edge_weights):
    s_col = _rowsum(x)
    s_flat = s_col.reshape(NPAD)

    dst3 = edge_index[1].reshape(NW, ROWS, CHUNK)
    t_part, c_part = _edge_call(s_flat, edge_index[0], dst3, edge_weights)

    info2d = _final(s_flat.reshape(NROWS, 128),
                    t_part.reshape(NC, NROWS, 128),
                    c_part.reshape(NC, NROWS, 128))
    return info2d.reshape(NPAD)[:N]


# speedup vs baseline: 47.6400x; 1.2905x over previous
import jax
import jax.numpy as jnp
from jax import lax
from jax.experimental import pallas as pl
from jax.experimental.pallas import tpu as pltpu
from jax.experimental.pallas import tpu_sc as plsc

N = 10000
E = 320000
D = 128

NPAD = 10240
NROWS = NPAD // 128
NC = 2
NS = 16
NW = NC * NS
EPT = E // NW
CHUNK = 80
ROWS = EPT // CHUNK
GRP = 5
NGRP = ROWS // GRP
SLICE = NPAD // NS


def _rowsum_body(x_ref, o_ref):
    o_ref[...] = jnp.sum(x_ref[...], axis=1).reshape(8, 128)


_rowsum = pl.pallas_call(
    _rowsum_body,
    grid=(NPAD // 1024,),
    in_specs=[pl.BlockSpec((1024, D), lambda i: (i, 0))],
    out_specs=pl.BlockSpec((8, 128), lambda i: (i, 0)),
    out_shape=jax.ShapeDtypeStruct((NROWS, 128), jnp.float32),
)


def _edge_body(s_hbm, ev_hbm, w_hbm, t_out, c_out,
               s_v, ev_v, w_v, msg_v, ones_v, zb_v,
               t_sh, c_sh, sem, scat_sem):
    cid = lax.axis_index("c")
    sid = lax.axis_index("s")
    wid = sid * NC + cid
    base = wid * EPT

    cps = [
        pltpu.async_copy(s_hbm, s_v, sem),
        pltpu.async_copy(ev_hbm.at[wid], ev_v, sem),
        pltpu.async_copy(w_hbm.at[pl.ds(base, EPT)], w_v, sem),
    ]

    for i in range(CHUNK // 16):
        ones_v[pl.ds(i * 16, 16)] = jnp.ones((16,), jnp.float32)

    def zbody(i, c):
        zb_v[pl.ds(i * 16, 16)] = jnp.zeros((16,), jnp.float32)
        return c
    lax.fori_loop(0, SLICE // 16, zbody, 0)
    pltpu.sync_copy(zb_v, t_sh.at[pl.ds(sid * SLICE, SLICE)])
    pltpu.sync_copy(zb_v, c_sh.at[pl.ds(sid * SLICE, SLICE)])
    for cp in cps:
        cp.wait()
    plsc.subcore_barrier()

    def scat_pair(j):
        row = pl.ds(j * CHUNK, CHUNK)
        c1 = pltpu.make_async_copy(msg_v.at[row], t_sh.at[ev_v.at[j, 1]],
                                   scat_sem)
        c2 = pltpu.make_async_copy(ones_v, c_sh.at[ev_v.at[j, 1]], scat_sem)
        return c1, c2

    def gbody(g, c):
        for r in range(GRP):
            j = g * GRP + r
            for k in range(CHUNK // 16):
                src16 = ev_v[j, 0, pl.ds(k * 16, 16)]
                vals = plsc.load_gather(
                    s_v, [lax.shift_right_logical(src16, 7),
                          lax.bitwise_and(src16, 127)])
                msg_v[pl.ds(j * CHUNK + k * 16, 16)] = (
                    vals * w_v[pl.ds(j * CHUNK + k * 16, 16)])
        for r in range(GRP):
            c1, c2 = scat_pair(g * GRP + r)
            c1.start(add=True)
            c2.start(add=True)

        @pl.when(g > 0)
        def _():
            for r in range(GRP):
                c1, c2 = scat_pair((g - 1) * GRP + r)
                c1.wait()
                c2.wait()
        return c
    lax.fori_loop(0, NGRP, gbody, 0)
    for r in range(GRP):
        c1, c2 = scat_pair((NGRP - 1) * GRP + r)
        c1.wait()
        c2.wait()
    plsc.subcore_barrier()

    off = sid * SLICE
    cpo1 = pltpu.async_copy(t_sh.at[pl.ds(off, SLICE)],
                            t_out.at[cid, pl.ds(off, SLICE)], sem)
    cpo2 = pltpu.async_copy(c_sh.at[pl.ds(off, SLICE)],
                            c_out.at[cid, pl.ds(off, SLICE)], sem)
    cpo1.wait()
    cpo2.wait()


_edge_call = pl.kernel(
    _edge_body,
    out_type=[jax.ShapeDtypeStruct((NC, NPAD), jnp.float32),
              jax.ShapeDtypeStruct((NC, NPAD), jnp.float32)],
    mesh=plsc.VectorSubcoreMesh(core_axis_name="c", subcore_axis_name="s",
                                num_cores=NC, num_subcores=NS),
    scratch_types=[
        pltpu.VMEM((NROWS, 128), jnp.float32),
        pltpu.VMEM((ROWS, 2, CHUNK), jnp.int32),
        pltpu.VMEM((EPT,), jnp.float32),
        pltpu.VMEM((EPT,), jnp.float32),
        pltpu.VMEM((CHUNK,), jnp.float32),
        pltpu.VMEM((SLICE,), jnp.float32),
        pltpu.VMEM_SHARED((NPAD,), jnp.float32),
        pltpu.VMEM_SHARED((NPAD,), jnp.float32),
        pltpu.SemaphoreType.DMA,
        pltpu.SemaphoreType.DMA,
    ],
    compiler_params=pltpu.CompilerParams(needs_layout_passes=False),
)


def _final_body(s_ref, t_ref, c_ref, o_ref):
    t = t_ref[0] + t_ref[1]
    c = jnp.maximum(c_ref[0] + c_ref[1], 1.0)
    o_ref[...] = jnp.abs(s_ref[...] - t / c)


_final = pl.pallas_call(
    _final_body,
    out_shape=jax.ShapeDtypeStruct((NROWS, 128), jnp.float32),
)


def kernel(x, edge_index, edge_weights):
    s2d = _rowsum(x)
    ev = edge_index.reshape(2, NW, ROWS, CHUNK).transpose(1, 2, 0, 3)

    t_part, c_part = _edge_call(s2d, ev, edge_weights)

    info2d = _final(s2d, t_part.reshape(NC, NROWS, 128),
                    c_part.reshape(NC, NROWS, 128))
    return info2d.reshape(NPAD)[:N]


# speedup vs baseline: 52.1489x; 1.0946x over previous
import jax
import jax.numpy as jnp
from jax import lax
from jax.experimental import pallas as pl
from jax.experimental.pallas import tpu as pltpu
from jax.experimental.pallas import tpu_sc as plsc

N = 10000
E = 320000
D = 128

NPAD = 10240
NROWS = NPAD // 128
NC = 2
NS = 16
NW = NC * NS
EPT = E // NW
CHUNK = 80
ROWS = EPT // CHUNK
GRP = 5
NGRP = ROWS // GRP
SLICE = NPAD // NS


def _rowsum_body(x_ref, o_ref):
    o_ref[...] = jnp.sum(x_ref[...], axis=1).reshape(8, 128)


_rowsum = pl.pallas_call(
    _rowsum_body,
    grid=(NPAD // 1024,),
    in_specs=[pl.BlockSpec((1024, D), lambda i: (i, 0))],
    out_specs=pl.BlockSpec((8, 128), lambda i: (i, 0)),
    out_shape=jax.ShapeDtypeStruct((NROWS, 128), jnp.float32),
)


def _edge_body(s_hbm, ev_hbm, w_hbm, t_out, c_out,
               s_v, src_v, dst_v, w_v, msg_v, ones_v, zb_v,
               t_sh, c_sh, sem, scat_sem):
    cid = lax.axis_index("c")
    sid = lax.axis_index("s")
    wid = sid * NC + cid
    base = wid * EPT

    cps = [
        pltpu.async_copy(s_hbm, s_v, sem),
        pltpu.async_copy(ev_hbm.at[0, wid], src_v, sem),
        pltpu.async_copy(ev_hbm.at[1, wid], dst_v, sem),
        pltpu.async_copy(w_hbm.at[pl.ds(base, EPT)], w_v, sem),
    ]

    for i in range(CHUNK // 16):
        ones_v[pl.ds(i * 16, 16)] = jnp.ones((16,), jnp.float32)

    def zbody(i, c):
        zb_v[pl.ds(i * 16, 16)] = jnp.zeros((16,), jnp.float32)
        return c
    lax.fori_loop(0, SLICE // 16, zbody, 0)
    pltpu.sync_copy(zb_v, t_sh.at[pl.ds(sid * SLICE, SLICE)])
    pltpu.sync_copy(zb_v, c_sh.at[pl.ds(sid * SLICE, SLICE)])
    for cp in cps:
        cp.wait()
    plsc.subcore_barrier()

    def scat_pair(j):
        row = pl.ds(j * CHUNK, CHUNK)
        c1 = pltpu.make_async_copy(msg_v.at[row], t_sh.at[dst_v.at[j]],
                                   scat_sem)
        c2 = pltpu.make_async_copy(ones_v, c_sh.at[dst_v.at[j]], scat_sem)
        return c1, c2

    def gbody(g, c):
        for r in range(GRP):
            j = g * GRP + r
            for k in range(CHUNK // 16):
                src16 = src_v[j, pl.ds(k * 16, 16)]
                vals = plsc.load_gather(
                    s_v, [lax.shift_right_logical(src16, 7),
                          lax.bitwise_and(src16, 127)])
                q = pl.ds(j * CHUNK + k * 16, 16)
                msg_v[q] = vals * w_v[q]
        for r in range(GRP):
            c1, c2 = scat_pair(g * GRP + r)
            c1.start(add=True)
            c2.start(add=True)

        @pl.when(g > 0)
        def _():
            for r in range(GRP):
                c1, c2 = scat_pair((g - 1) * GRP + r)
                c1.wait()
                c2.wait()
        return c
    lax.fori_loop(0, NGRP, gbody, 0)
    for r in range(GRP):
        c1, c2 = scat_pair((NGRP - 1) * GRP + r)
        c1.wait()
        c2.wait()
    plsc.subcore_barrier()

    off = sid * SLICE
    cpo1 = pltpu.async_copy(t_sh.at[pl.ds(off, SLICE)],
                            t_out.at[cid, pl.ds(off, SLICE)], sem)
    cpo2 = pltpu.async_copy(c_sh.at[pl.ds(off, SLICE)],
                            c_out.at[cid, pl.ds(off, SLICE)], sem)
    cpo1.wait()
    cpo2.wait()


_edge_call = pl.kernel(
    _edge_body,
    out_type=[jax.ShapeDtypeStruct((NC, NPAD), jnp.float32),
              jax.ShapeDtypeStruct((NC, NPAD), jnp.float32)],
    mesh=plsc.VectorSubcoreMesh(core_axis_name="c", subcore_axis_name="s",
                                num_cores=NC, num_subcores=NS),
    scratch_types=[
        pltpu.VMEM((NROWS, 128), jnp.float32),
        pltpu.VMEM((ROWS, CHUNK), jnp.int32),
        pltpu.VMEM((ROWS, CHUNK), jnp.int32),
        pltpu.VMEM((EPT,), jnp.float32),
        pltpu.VMEM((EPT,), jnp.float32),
        pltpu.VMEM((CHUNK,), jnp.float32),
        pltpu.VMEM((SLICE,), jnp.float32),
        pltpu.VMEM_SHARED((NPAD,), jnp.float32),
        pltpu.VMEM_SHARED((NPAD,), jnp.float32),
        pltpu.SemaphoreType.DMA,
        pltpu.SemaphoreType.DMA,
    ],
    compiler_params=pltpu.CompilerParams(needs_layout_passes=False),
)


def _final_body(s_ref, t_ref, c_ref, o_ref):
    t = t_ref[0] + t_ref[1]
    c = jnp.maximum(c_ref[0] + c_ref[1], 1.0)
    o_ref[...] = jnp.abs(s_ref[...] - t / c)


_final = pl.pallas_call(
    _final_body,
    out_shape=jax.ShapeDtypeStruct((NROWS, 128), jnp.float32),
)


def kernel(x, edge_index, edge_weights):
    s2d = _rowsum(x)
    ev = edge_index.reshape(2, NW, ROWS, CHUNK)

    t_part, c_part = _edge_call(s2d, ev, edge_weights)

    info2d = _final(s2d, t_part.reshape(NC, NROWS, 128),
                    c_part.reshape(NC, NROWS, 128))
    return info2d.reshape(NPAD)[:N]


# speedup vs baseline: 60.3274x; 1.1568x over previous
import jax
import jax.numpy as jnp
from jax import lax
from jax.experimental import pallas as pl
from jax.experimental.pallas import tpu as pltpu
from jax.experimental.pallas import tpu_sc as plsc

N = 10000
E = 320000
D = 128

NPAD = 10240
NROWS = NPAD // 128
NC = 2
NS = 16
NW = NC * NS
EROWS = E // 128
RPT = EROWS // NW
REM = EROWS - NW * RPT
RMAX = RPT + 1
SLICE = NPAD // NS


def _rowsum_body(x_ref, o_ref):
    o_ref[...] = jnp.sum(x_ref[...], axis=1).reshape(8, 128)


_rowsum = pl.pallas_call(
    _rowsum_body,
    grid=(NPAD // 1024,),
    in_specs=[pl.BlockSpec((1024, D), lambda i: (i, 0))],
    out_specs=pl.BlockSpec((8, 128), lambda i: (i, 0)),
    out_shape=jax.ShapeDtypeStruct((NROWS, 128), jnp.float32),
)


def _edge_body(s_hbm, sd_hbm, w_hbm, t_out, c_out,
               s_v, sd_v, w_v, msg_v, ones_v, zb_v,
               t_sh, c_sh, sem, scat_sem):
    cid = lax.axis_index("c")
    sid = lax.axis_index("s")
    wid = sid * NC + cid

    r_lo = RPT * wid + jnp.minimum(wid, REM)
    nrows = RPT + jnp.where(wid < REM, 1, 0)
    start = jnp.minimum(r_lo, EROWS - RMAX)
    roff = r_lo - start

    cps = [
        pltpu.async_copy(s_hbm, s_v, sem),
        pltpu.async_copy(sd_hbm.at[pl.ds(start, RMAX)], sd_v, sem),
        pltpu.async_copy(w_hbm.at[pl.ds(start * 128, RMAX * 128)], w_v, sem),
    ]

    for i in range(128 // 16):
        ones_v[pl.ds(i * 16, 16)] = jnp.ones((16,), jnp.float32)

    def zbody(i, c):
        zb_v[pl.ds(i * 16, 16)] = jnp.zeros((16,), jnp.float32)
        return c
    lax.fori_loop(0, SLICE // 16, zbody, 0)
    pltpu.sync_copy(zb_v, t_sh.at[pl.ds(sid * SLICE, SLICE)])
    pltpu.sync_copy(zb_v, c_sh.at[pl.ds(sid * SLICE, SLICE)])
    for cp in cps:
        cp.wait()
    plsc.subcore_barrier()

    def scat_pair(r):
        c1 = pltpu.make_async_copy(msg_v.at[pl.ds(r * 128, 128)],
                                   t_sh.at[sd_v.at[r, 1]], scat_sem)
        c2 = pltpu.make_async_copy(ones_v, c_sh.at[sd_v.at[r, 1]], scat_sem)
        return c1, c2

    def rbody(r, c):
        for k in range(8):
            src16 = sd_v[r, 0, pl.ds(k * 16, 16)]
            vals = plsc.load_gather(
                s_v, [lax.shift_right_logical(src16, 7),
                      lax.bitwise_and(src16, 127)])
            q = pl.ds(r * 128 + k * 16, 16)
            msg_v[q] = vals * w_v[q]
        c1, c2 = scat_pair(r)
        c1.start(add=True)
        c2.start(add=True)

        @pl.when(r > roff + 1)
        def _():
            p1, p2 = scat_pair(r - 2)
            p1.wait()
            p2.wait()
        return c
    lax.fori_loop(roff, roff + nrows, rbody, 0)
    for back in (2, 1):
        f1, f2 = scat_pair(roff + nrows - back)
        f1.wait()
        f2.wait()
    plsc.subcore_barrier()

    off = sid * SLICE
    cpo1 = pltpu.async_copy(t_sh.at[pl.ds(off, SLICE)],
                            t_out.at[cid, pl.ds(off, SLICE)], sem)
    cpo2 = pltpu.async_copy(c_sh.at[pl.ds(off, SLICE)],
                            c_out.at[cid, pl.ds(off, SLICE)], sem)
    cpo1.wait()
    cpo2.wait()


_edge_call = pl.kernel(
    _edge_body,
    out_type=[jax.ShapeDtypeStruct((NC, NPAD), jnp.float32),
              jax.ShapeDtypeStruct((NC, NPAD), jnp.float32)],
    mesh=plsc.VectorSubcoreMesh(core_axis_name="c", subcore_axis_name="s",
                                num_cores=NC, num_subcores=NS),
    scratch_types=[
        pltpu.VMEM((NROWS, 128), jnp.float32),
        pltpu.VMEM((RMAX, 2, 128), jnp.int32),
        pltpu.VMEM((RMAX * 128,), jnp.float32),
        pltpu.VMEM((RMAX * 128,), jnp.float32),
        pltpu.VMEM((128,), jnp.float32),
        pltpu.VMEM((SLICE,), jnp.float32),
        pltpu.VMEM_SHARED((NPAD,), jnp.float32),
        pltpu.VMEM_SHARED((NPAD,), jnp.float32),
        pltpu.SemaphoreType.DMA,
        pltpu.SemaphoreType.DMA,
    ],
    compiler_params=pltpu.CompilerParams(needs_layout_passes=False),
)


def _final_body(s_ref, t_ref, c_ref, o_ref):
    t = t_ref[0] + t_ref[1]
    c = jnp.maximum(c_ref[0] + c_ref[1], 1.0)
    o_ref[...] = jnp.abs(s_ref[...] - t / c)


_final = pl.pallas_call(
    _final_body,
    out_shape=jax.ShapeDtypeStruct((NROWS, 128), jnp.float32),
)


def kernel(x, edge_index, edge_weights):
    s2d = _rowsum(x)
    sd = edge_index.reshape(2, EROWS, 128).transpose(1, 0, 2)

    t_part, c_part = _edge_call(s2d, sd, edge_weights)

    info2d = _final(s2d, t_part.reshape(NC, NROWS, 128),
                    c_part.reshape(NC, NROWS, 128))
    return info2d.reshape(NPAD)[:N]


# speedup vs baseline: 66.6960x; 1.1056x over previous
import jax
import jax.numpy as jnp
from jax import lax
from jax.experimental import pallas as pl
from jax.experimental.pallas import tpu as pltpu
from jax.experimental.pallas import tpu_sc as plsc

N = 10000
E = 320000
D = 128

NPAD = 10240
NROWS = NPAD // 128
NC = 2
NS = 16
NW = NC * NS
EROWS = E // 128
RPT = EROWS // NW
REM = EROWS - NW * RPT
RMAX = RPT + 1
SLICE = NPAD // NS


def _rowsum_body(x_ref, o_ref):
    o_ref[...] = jnp.sum(x_ref[...], axis=1).reshape(16, 128)


_rowsum = pl.pallas_call(
    _rowsum_body,
    grid=(NPAD // 2048,),
    in_specs=[pl.BlockSpec((2048, D), lambda i: (i, 0))],
    out_specs=pl.BlockSpec((16, 128), lambda i: (i, 0)),
    out_shape=jax.ShapeDtypeStruct((NROWS, 128), jnp.float32),
)


def _edge_body(s_hbm, sd_hbm, w_hbm, t0_out, t1_out, c0_out, c1_out,
               s_v, sd_v, w_v, msg_v, ones_v, zb_v,
               t_sh, c_sh, sem, scat_sem):
    cid = lax.axis_index("c")
    sid = lax.axis_index("s")
    wid = sid * NC + cid

    r_lo = RPT * wid + jnp.minimum(wid, REM)
    nrows = RPT + jnp.where(wid < REM, 1, 0)
    start = jnp.minimum(r_lo, EROWS - RMAX)
    roff = r_lo - start

    cps = [
        pltpu.async_copy(s_hbm, s_v, sem),
        pltpu.async_copy(sd_hbm.at[pl.ds(start, RMAX)], sd_v, sem),
        pltpu.async_copy(w_hbm.at[pl.ds(start * 128, RMAX * 128)], w_v, sem),
    ]

    for i in range(128 // 16):
        ones_v[pl.ds(i * 16, 16)] = jnp.ones((16,), jnp.float32)

    def zbody(i, c):
        zb_v[pl.ds(i * 16, 16)] = jnp.zeros((16,), jnp.float32)
        return c
    lax.fori_loop(0, SLICE // 16, zbody, 0)
    pltpu.sync_copy(zb_v, t_sh.at[pl.ds(sid * SLICE, SLICE)])
    pltpu.sync_copy(zb_v, c_sh.at[pl.ds(sid * SLICE, SLICE)])
    for cp in cps:
        cp.wait()
    plsc.subcore_barrier()

    def scat_pair(r):
        c1 = pltpu.make_async_copy(msg_v.at[pl.ds(r * 128, 128)],
                                   t_sh.at[sd_v.at[r, 1]], scat_sem)
        c2 = pltpu.make_async_copy(ones_v, c_sh.at[sd_v.at[r, 1]], scat_sem)
        return c1, c2

    def rbody(r, c):
        for k in range(8):
            src16 = sd_v[r, 0, pl.ds(k * 16, 16)]
            vals = plsc.load_gather(
                s_v, [lax.shift_right_logical(src16, 7),
                      lax.bitwise_and(src16, 127)])
            q = pl.ds(r * 128 + k * 16, 16)
            msg_v[q] = vals * w_v[q]
        c1, c2 = scat_pair(r)
        c1.start(add=True)
        c2.start(add=True)

        @pl.when(r > roff + 1)
        def _():
            p1, p2 = scat_pair(r - 2)
            p1.wait()
            p2.wait()
        return c
    lax.fori_loop(roff, roff + nrows, rbody, 0)
    for back in (2, 1):
        f1, f2 = scat_pair(roff + nrows - back)
        f1.wait()
        f2.wait()
    plsc.subcore_barrier()

    off = sid * SLICE

    @pl.when(cid == 0)
    def _():
        cpo1 = pltpu.async_copy(t_sh.at[pl.ds(off, SLICE)],
                                t0_out.at[pl.ds(off, SLICE)], sem)
        cpo2 = pltpu.async_copy(c_sh.at[pl.ds(off, SLICE)],
                                c0_out.at[pl.ds(off, SLICE)], sem)
        cpo1.wait()
        cpo2.wait()

    @pl.when(cid == 1)
    def _():
        cpo1 = pltpu.async_copy(t_sh.at[pl.ds(off, SLICE)],
                                t1_out.at[pl.ds(off, SLICE)], sem)
        cpo2 = pltpu.async_copy(c_sh.at[pl.ds(off, SLICE)],
                                c1_out.at[pl.ds(off, SLICE)], sem)
        cpo1.wait()
        cpo2.wait()


_edge_call = pl.kernel(
    _edge_body,
    out_type=[jax.ShapeDtypeStruct((NPAD,), jnp.float32)] * 4,
    mesh=plsc.VectorSubcoreMesh(core_axis_name="c", subcore_axis_name="s",
                                num_cores=NC, num_subcores=NS),
    scratch_types=[
        pltpu.VMEM((NROWS, 128), jnp.float32),
        pltpu.VMEM((RMAX, 2, 128), jnp.int32),
        pltpu.VMEM((RMAX * 128,), jnp.float32),
        pltpu.VMEM((RMAX * 128,), jnp.float32),
        pltpu.VMEM((128,), jnp.float32),
        pltpu.VMEM((SLICE,), jnp.float32),
        pltpu.VMEM_SHARED((NPAD,), jnp.float32),
        pltpu.VMEM_SHARED((NPAD,), jnp.float32),
        pltpu.SemaphoreType.DMA,
        pltpu.SemaphoreType.DMA,
    ],
    compiler_params=pltpu.CompilerParams(needs_layout_passes=False),
)


def _final_body(s_ref, t0_ref, t1_ref, c0_ref, c1_ref, o_ref):
    t = t0_ref[...] + t1_ref[...]
    c = jnp.maximum(c0_ref[...] + c1_ref[...], 1.0)
    o_ref[...] = jnp.abs(s_ref[...] - t / c)


_final = pl.pallas_call(
    _final_body,
    out_shape=jax.ShapeDtypeStruct((NROWS, 128), jnp.float32),
)


def kernel(x, edge_index, edge_weights):
    s2d = _rowsum(x)
    sd = edge_index.reshape(2, EROWS, 128).transpose(1, 0, 2)

    t0, t1, c0, c1 = _edge_call(s2d, sd, edge_weights)

    info2d = _final(s2d, t0.reshape(NROWS, 128), t1.reshape(NROWS, 128),
                    c0.reshape(NROWS, 128), c1.reshape(NROWS, 128))
    return info2d.reshape(NPAD)[:N]


# speedup vs baseline: 67.3825x; 1.0103x over previous
import jax
import jax.numpy as jnp
from jax import lax
from jax.experimental import pallas as pl
from jax.experimental.pallas import tpu as pltpu
from jax.experimental.pallas import tpu_sc as plsc

N = 10000
E = 320000
D = 128

NPAD = 10240
NROWS = NPAD // 128
NC = 2
NS = 16
NW = NC * NS
EROWS = E // 128
RPT = EROWS // NW
REM = EROWS - NW * RPT
RMAX = RPT + 1
SLICE = NPAD // NS


def _rowsum_body(x_ref, o_ref):
    o_ref[...] = jnp.sum(x_ref[...], axis=1).reshape(16, 128)


_rowsum = pl.pallas_call(
    _rowsum_body,
    grid=(NPAD // 2048,),
    in_specs=[pl.BlockSpec((2048, D), lambda i: (i, 0))],
    out_specs=pl.BlockSpec((16, 128), lambda i: (i, 0)),
    out_shape=jax.ShapeDtypeStruct((NROWS, 128), jnp.float32),
)


def _edge_body(s_hbm, sd_hbm, w_hbm, t0_out, t1_out, c0_out, c1_out,
               s_v, sd_v, w_v, msg_v, ones_v, zb_v,
               t_sh, c_sh, sem, scat_sem):
    cid = lax.axis_index("c")
    sid = lax.axis_index("s")
    wid = sid * NC + cid

    r_lo = RPT * wid + jnp.minimum(wid, REM)
    nrows = RPT + jnp.where(wid < REM, 1, 0)
    start = jnp.minimum(r_lo, EROWS - RMAX)
    roff = r_lo - start

    cps = [
        pltpu.async_copy(s_hbm, s_v, sem),
        pltpu.async_copy(sd_hbm.at[pl.ds(start, RMAX)], sd_v, sem),
        pltpu.async_copy(w_hbm.at[pl.ds(start * 128, RMAX * 128)], w_v, sem),
    ]

    for i in range(128 // 16):
        ones_v[pl.ds(i * 16, 16)] = jnp.ones((16,), jnp.float32)

    def zbody(i, c):
        zb_v[pl.ds(i * 16, 16)] = jnp.zeros((16,), jnp.float32)
        return c
    lax.fori_loop(0, SLICE // 16, zbody, 0)
    pltpu.sync_copy(zb_v, t_sh.at[pl.ds(sid * SLICE, SLICE)])
    pltpu.sync_copy(zb_v, c_sh.at[pl.ds(sid * SLICE, SLICE)])
    for cp in cps:
        cp.wait()
    plsc.subcore_barrier()

    def scat_pair(r):
        c1 = pltpu.make_async_copy(msg_v.at[pl.ds(r * 128, 128)],
                                   t_sh.at[sd_v.at[r, 1]], scat_sem)
        c2 = pltpu.make_async_copy(ones_v, c_sh.at[sd_v.at[r, 1]], scat_sem)
        return c1, c2

    def rbody(r, c):
        srcs = [sd_v[r, 0, pl.ds(k * 16, 16)] for k in range(8)]
        vals = [plsc.load_gather(s_v, [lax.shift_right_logical(s16, 7),
                                       lax.bitwise_and(s16, 127)])
                for s16 in srcs]
        ws = [w_v[pl.ds(r * 128 + k * 16, 16)] for k in range(8)]
        for k in range(8):
            msg_v[pl.ds(r * 128 + k * 16, 16)] = vals[k] * ws[k]
        c1, c2 = scat_pair(r)
        c1.start(add=True)
        c2.start(add=True)

        @pl.when(r > roff + 1)
        def _():
            p1, p2 = scat_pair(r - 2)
            p1.wait()
            p2.wait()
        return c
    lax.fori_loop(roff, roff + nrows, rbody, 0)
    for back in (2, 1):
        f1, f2 = scat_pair(roff + nrows - back)
        f1.wait()
        f2.wait()
    plsc.subcore_barrier()

    off = sid * SLICE

    @pl.when(cid == 0)
    def _():
        cpo1 = pltpu.async_copy(t_sh.at[pl.ds(off, SLICE)],
                                t0_out.at[pl.ds(off, SLICE)], sem)
        cpo2 = pltpu.async_copy(c_sh.at[pl.ds(off, SLICE)],
                                c0_out.at[pl.ds(off, SLICE)], sem)
        cpo1.wait()
        cpo2.wait()

    @pl.when(cid == 1)
    def _():
        cpo1 = pltpu.async_copy(t_sh.at[pl.ds(off, SLICE)],
                                t1_out.at[pl.ds(off, SLICE)], sem)
        cpo2 = pltpu.async_copy(c_sh.at[pl.ds(off, SLICE)],
                                c1_out.at[pl.ds(off, SLICE)], sem)
        cpo1.wait()
        cpo2.wait()


_edge_call = pl.kernel(
    _edge_body,
    out_type=[jax.ShapeDtypeStruct((NPAD,), jnp.float32)] * 4,
    mesh=plsc.VectorSubcoreMesh(core_axis_name="c", subcore_axis_name="s",
                                num_cores=NC, num_subcores=NS),
    scratch_types=[
        pltpu.VMEM((NROWS, 128), jnp.float32),
        pltpu.VMEM((RMAX, 2, 128), jnp.int32),
        pltpu.VMEM((RMAX * 128,), jnp.float32),
        pltpu.VMEM((RMAX * 128,), jnp.float32),
        pltpu.VMEM((128,), jnp.float32),
        pltpu.VMEM((SLICE,), jnp.float32),
        pltpu.VMEM_SHARED((NPAD,), jnp.float32),
        pltpu.VMEM_SHARED((NPAD,), jnp.float32),
        pltpu.SemaphoreType.DMA,
        pltpu.SemaphoreType.DMA,
    ],
    compiler_params=pltpu.CompilerParams(needs_layout_passes=False),
)


def _final_body(s_ref, t0_ref, t1_ref, c0_ref, c1_ref, o_ref):
    t = t0_ref[...] + t1_ref[...]
    c = jnp.maximum(c0_ref[...] + c1_ref[...], 1.0)
    o_ref[...] = jnp.abs(s_ref[...] - t / c)


_final = pl.pallas_call(
    _final_body,
    out_shape=jax.ShapeDtypeStruct((NROWS, 128), jnp.float32),
)


def kernel(x, edge_index, edge_weights):
    s2d = _rowsum(x)
    sd = edge_index.reshape(2, EROWS, 128).transpose(1, 0, 2)

    t0, t1, c0, c1 = _edge_call(s2d, sd, edge_weights)

    info2d = _final(s2d, t0.reshape(NROWS, 128), t1.reshape(NROWS, 128),
                    c0.reshape(NROWS, 128), c1.reshape(NROWS, 128))
    return info2d.reshape(NPAD)[:N]


# speedup vs baseline: 69.9899x; 1.0387x over previous
import jax
import jax.numpy as jnp
from jax import lax
from jax.experimental import pallas as pl
from jax.experimental.pallas import tpu as pltpu
from jax.experimental.pallas import tpu_sc as plsc

N = 10000
E = 320000
D = 128

NPAD = 10240
NROWS = NPAD // 128
NC = 2
NS = 16
NW = NC * NS
EROWS = E // 128
RPT = EROWS // NW
REM = EROWS - NW * RPT
RMAX = RPT + 1
GRP = 6
SLICE = NPAD // NS


def _rowsum_body(x_ref, o_ref):
    o_ref[...] = jnp.sum(x_ref[...], axis=1).reshape(16, 128)


_rowsum = pl.pallas_call(
    _rowsum_body,
    grid=(NPAD // 2048,),
    in_specs=[pl.BlockSpec((2048, D), lambda i: (i, 0))],
    out_specs=pl.BlockSpec((16, 128), lambda i: (i, 0)),
    out_shape=jax.ShapeDtypeStruct((NROWS, 128), jnp.float32),
)


def _edge_body(s_hbm, sd_hbm, w_hbm, t0_out, t1_out, c0_out, c1_out,
               s_v, sd_v, w_v, msg_v, ones_v, zb_v,
               t_sh, c_sh, sem, scat_sem):
    cid = lax.axis_index("c")
    sid = lax.axis_index("s")
    wid = sid * NC + cid

    r_lo = RPT * wid + jnp.minimum(wid, REM)
    start = jnp.minimum(r_lo, EROWS - RMAX)
    roff = r_lo - start

    cps = [
        pltpu.async_copy(s_hbm, s_v, sem),
        pltpu.async_copy(sd_hbm.at[pl.ds(start, RMAX)], sd_v, sem),
        pltpu.async_copy(w_hbm.at[pl.ds(start * 128, RMAX * 128)], w_v, sem),
    ]

    for i in range(128 // 16):
        ones_v[pl.ds(i * 16, 16)] = jnp.ones((16,), jnp.float32)

    def zbody(i, c):
        zb_v[pl.ds(i * 16, 16)] = jnp.zeros((16,), jnp.float32)
        return c
    lax.fori_loop(0, SLICE // 16, zbody, 0)
    pltpu.sync_copy(zb_v, t_sh.at[pl.ds(sid * SLICE, SLICE)])
    pltpu.sync_copy(zb_v, c_sh.at[pl.ds(sid * SLICE, SLICE)])
    for cp in cps:
        cp.wait()
    plsc.subcore_barrier()

    def scat_pair(r):
        c1 = pltpu.make_async_copy(msg_v.at[pl.ds(r * 128, 128)],
                                   t_sh.at[sd_v.at[r, 1]], scat_sem)
        c2 = pltpu.make_async_copy(ones_v, c_sh.at[sd_v.at[r, 1]], scat_sem)
        return c1, c2

    def do_row(r):
        srcs = [sd_v[r, 0, pl.ds(k * 16, 16)] for k in range(8)]
        vals = [plsc.load_gather(s_v, [lax.shift_right_logical(s16, 7),
                                       lax.bitwise_and(s16, 127)])
                for s16 in srcs]
        ws = [w_v[pl.ds(r * 128 + k * 16, 16)] for k in range(8)]
        for k in range(8):
            msg_v[pl.ds(r * 128 + k * 16, 16)] = vals[k] * ws[k]
        c1, c2 = scat_pair(r)
        c1.start(add=True)
        c2.start(add=True)

    def gbody(g, c):
        for i in range(GRP):
            do_row(roff + g * GRP + i)

        @pl.when(g > 0)
        def _():
            for i in range(GRP):
                p1, p2 = scat_pair(roff + (g - 1) * GRP + i)
                p1.wait()
                p2.wait()
        return c
    lax.fori_loop(0, RPT // GRP, gbody, 0)
    for i in range(GRP):
        f1, f2 = scat_pair(roff + RPT - GRP + i)
        f1.wait()
        f2.wait()

    @pl.when(wid < REM)
    def _():
        do_row(roff + RPT)
        e1, e2 = scat_pair(roff + RPT)
        e1.wait()
        e2.wait()
    plsc.subcore_barrier()

    off = sid * SLICE

    @pl.when(cid == 0)
    def _():
        cpo1 = pltpu.async_copy(t_sh.at[pl.ds(off, SLICE)],
                                t0_out.at[pl.ds(off, SLICE)], sem)
        cpo2 = pltpu.async_copy(c_sh.at[pl.ds(off, SLICE)],
                                c0_out.at[pl.ds(off, SLICE)], sem)
        cpo1.wait()
        cpo2.wait()

    @pl.when(cid == 1)
    def _():
        cpo1 = pltpu.async_copy(t_sh.at[pl.ds(off, SLICE)],
                                t1_out.at[pl.ds(off, SLICE)], sem)
        cpo2 = pltpu.async_copy(c_sh.at[pl.ds(off, SLICE)],
                                c1_out.at[pl.ds(off, SLICE)], sem)
        cpo1.wait()
        cpo2.wait()


_edge_call = pl.kernel(
    _edge_body,
    out_type=[jax.ShapeDtypeStruct((NPAD,), jnp.float32)] * 4,
    mesh=plsc.VectorSubcoreMesh(core_axis_name="c", subcore_axis_name="s",
                                num_cores=NC, num_subcores=NS),
    scratch_types=[
        pltpu.VMEM((NROWS, 128), jnp.float32),
        pltpu.VMEM((RMAX, 2, 128), jnp.int32),
        pltpu.VMEM((RMAX * 128,), jnp.float32),
        pltpu.VMEM((RMAX * 128,), jnp.float32),
        pltpu.VMEM((128,), jnp.float32),
        pltpu.VMEM((SLICE,), jnp.float32),
        pltpu.VMEM_SHARED((NPAD,), jnp.float32),
        pltpu.VMEM_SHARED((NPAD,), jnp.float32),
        pltpu.SemaphoreType.DMA,
        pltpu.SemaphoreType.DMA,
    ],
    compiler_params=pltpu.CompilerParams(needs_layout_passes=False),
)


def _final_body(s_ref, t0_ref, t1_ref, c0_ref, c1_ref, o_ref):
    t = t0_ref[...] + t1_ref[...]
    c = jnp.maximum(c0_ref[...] + c1_ref[...], 1.0)
    o_ref[...] = jnp.abs(s_ref[...] - t / c)


_final = pl.pallas_call(
    _final_body,
    out_shape=jax.ShapeDtypeStruct((NROWS, 128), jnp.float32),
)


def kernel(x, edge_index, edge_weights):
    s2d = _rowsum(x)
    sd = edge_index.reshape(2, EROWS, 128).transpose(1, 0, 2)

    t0, t1, c0, c1 = _edge_call(s2d, sd, edge_weights)

    info2d = _final(s2d, t0.reshape(NROWS, 128), t1.reshape(NROWS, 128),
                    c0.reshape(NROWS, 128), c1.reshape(NROWS, 128))
    return info2d.reshape(NPAD)[:N]
